# baseline (speedup 1.0000x reference)
import functools

import jax
import jax.numpy as jnp
from jax import lax
from jax.experimental import pallas as pl
from jax.experimental.pallas import tpu as pltpu
from jax.experimental.pallas import tpu_sc as plsc

N = 100000
H = 512
NUM_GRAPHS = 256
BN = 2000
NBLK = N // BN

NC = 2
NS = 16
HC = H // NC
N_PAD = 100352
RPT = N_PAD // NS
CH = 112
NCH = RPT // CH
ACC_R = 264
LANES = 16


def _tc1(v_ref, w1_ref, b1_ref, h_ref):
    vb = v_ref[...].astype(jnp.bfloat16)
    h = lax.dot_general(vb, w1_ref[...], (((1,), (1,)), ((), ())),
                        preferred_element_type=jnp.float32)
    h_ref[...] = jnp.maximum(h + b1_ref[...], 0.0)


def _sc_pool(h_hbm, ids_hbm, zeros_hbm, out_hbm, rows_v, idx_v, acc_v):
    c = lax.axis_index("c")
    s = lax.axis_index("s")
    base = s * RPT
    iota = lax.iota(jnp.int32, LANES)

    pltpu.sync_copy(zeros_hbm, acc_v)

    @pl.loop(0, NCH)
    def _chunk(k):
        pltpu.sync_copy(h_hbm.at[pl.ds(base + k * CH, CH),
                                 pl.ds(c * HC, HC)], rows_v)
        pltpu.sync_copy(ids_hbm.at[pl.ds(base + k * CH, CH)], idx_v)

        @pl.loop(0, CH)
        def _row(r):
            rsp = jnp.full((LANES,), r, jnp.int32)
            idv = plsc.load_gather(idx_v, [rsp])
            for j in range(HC // LANES):
                piece = plsc.load_gather(rows_v, [rsp, iota + j * LANES])
                plsc.addupdate_scatter(acc_v, [idv, iota + j * LANES], piece)

    pltpu.sync_copy(acc_v.at[pl.ds(0, NUM_GRAPHS)],
                    out_hbm.at[s, pl.ds(0, NUM_GRAPHS), pl.ds(c * HC, HC)])


def _tc2(parts_ref, w2_ref, b2_ref, out_ref):
    pooled = jnp.sum(parts_ref[...], axis=0)
    o = lax.dot_general(pooled, w2_ref[...], (((1,), (1,)), ((), ())),
                        preferred_element_type=jnp.float32)
    out_ref[...] = jnp.maximum(o + b2_ref[...], 0.0)


@jax.jit
def kernel(v, W1, b1, W2, b2, batch):
    W1b = W1.astype(jnp.bfloat16)
    b1r = b1.reshape(1, H)
    b2r = b2.reshape(1, H)
    ids_pad = jnp.concatenate(
        [batch.astype(jnp.int32),
         jnp.full((N_PAD - N,), NUM_GRAPHS, jnp.int32)])
    zeros = jnp.zeros((ACC_R, HC), jnp.float32)

    h = pl.pallas_call(
        _tc1,
        grid=(NBLK,),
        in_specs=[
            pl.BlockSpec((BN, 3 * H), lambda i: (i, 0)),
            pl.BlockSpec((H, 3 * H), lambda i: (0, 0)),
            pl.BlockSpec((1, H), lambda i: (0, 0)),
        ],
        out_specs=pl.BlockSpec((BN, H), lambda i: (i, 0)),
        out_shape=jax.ShapeDtypeStruct((N_PAD, H), jnp.float32),
        compiler_params=pltpu.CompilerParams(
            dimension_semantics=("parallel",)),
    )(v, W1b, b1r)

    mesh = plsc.VectorSubcoreMesh(core_axis_name="c", subcore_axis_name="s")
    parts = pl.kernel(
        _sc_pool,
        out_type=jax.ShapeDtypeStruct((NS, NUM_GRAPHS, H), jnp.float32),
        mesh=mesh,
        compiler_params=pltpu.CompilerParams(needs_layout_passes=False),
        scratch_types=[
            pltpu.VMEM((CH, HC), jnp.float32),
            pltpu.VMEM((CH,), jnp.int32),
            pltpu.VMEM((ACC_R, HC), jnp.float32),
        ],
    )(h, ids_pad, zeros)

    out = pl.pallas_call(
        _tc2,
        in_specs=[
            pl.BlockSpec((NS, NUM_GRAPHS, H), lambda: (0, 0, 0)),
            pl.BlockSpec((H, H), lambda: (0, 0)),
            pl.BlockSpec((1, H), lambda: (0, 0)),
        ],
        out_specs=pl.BlockSpec((NUM_GRAPHS, H), lambda: (0, 0)),
        out_shape=jax.ShapeDtypeStruct((NUM_GRAPHS, H), jnp.float32),
    )(parts, W2, b2r)
    return out
